# baseline (speedup 1.0000x reference)
import functools
import math

import jax
import jax.numpy as jnp
from jax import lax
from jax.experimental import pallas as pl
from jax.experimental.pallas import tpu as pltpu
from jax.experimental.pallas import tpu_sc as plsc

N_ROWS = 2048
SIZE = 32000
PAD = 0
EPS = 0.1 / (SIZE - 2)
CONF = 0.9
C0 = 0.1 * math.log(EPS) + CONF * math.log(CONF)

NC = 2
NS = 16
NW = NC * NS
RPW = N_ROWS // NW
LANES = 16

CROWS = 8
NCH = N_ROWS // CROWS
NBUF = 16


def _sc_body(x_flat_hbm, tgt_hbm, out_hbm, tgt_v, idx_v, idx0_v, g_v, x0_v,
             acc_v, sem):
    wid = lax.axis_index("s") * NC + lax.axis_index("c")
    base = wid * RPW
    pltpu.sync_copy(tgt_hbm.at[pl.ds(base, RPW)], tgt_v)
    for j in range(RPW // LANES):
        t = tgt_v[pl.ds(j * LANES, LANES)]
        row = base + (j * LANES + lax.iota(jnp.int32, LANES))
        idx_v[pl.ds(j * LANES, LANES)] = row * SIZE + t
        idx0_v[pl.ds(j * LANES, LANES)] = row * SIZE
    pltpu.async_copy(x_flat_hbm.at[idx_v], g_v, sem).wait()
    pltpu.async_copy(x_flat_hbm.at[idx0_v], x0_v, sem).wait()
    acc = jnp.zeros((LANES,), jnp.float32)
    zero = jnp.zeros((LANES,), jnp.float32)
    for j in range(RPW // LANES):
        t = tgt_v[pl.ds(j * LANES, LANES)]
        g = g_v[pl.ds(j * LANES, LANES)]
        x0 = x0_v[pl.ds(j * LANES, LANES)]
        contrib = (EPS - CONF) * g + EPS * x0 + C0
        acc = acc + jnp.where(t != PAD, contrib, zero)
    acc_v[...] = acc
    pltpu.sync_copy(acc_v, out_hbm.at[wid])


@functools.cache
def _sc_gather():
    return pl.kernel(
        _sc_body,
        mesh=plsc.VectorSubcoreMesh(core_axis_name="c", subcore_axis_name="s"),
        out_type=jax.ShapeDtypeStruct((NW, LANES), jnp.float32),
        scratch_types=[
            pltpu.VMEM((RPW,), jnp.int32),
            pltpu.VMEM((RPW,), jnp.int32),
            pltpu.VMEM((RPW,), jnp.int32),
            pltpu.VMEM((RPW,), jnp.float32),
            pltpu.VMEM((RPW,), jnp.float32),
            pltpu.VMEM((LANES,), jnp.float32),
            pltpu.SemaphoreType.DMA,
        ],
    )


def _tc_body(x_hbm, m_ref, sc_ref, out_ref, bufs, sems):
    def issue(c, k):
        pltpu.make_async_copy(
            x_hbm.at[pl.ds(c * CROWS, CROWS), :], bufs.at[k], sems.at[k]
        ).start()

    for k in range(NBUF):
        issue(k, k)

    def outer(o, acc):
        for k in range(NBUF):
            c = o * NBUF + k
            pltpu.make_async_copy(
                x_hbm.at[pl.ds(c * CROWS, CROWS), :], bufs.at[k], sems.at[k]
            ).wait()
            rs = jnp.sum(bufs[k], axis=1, keepdims=True)
            acc = acc + rs * m_ref[pl.ds(c * CROWS, CROWS), :]
            nc = c + NBUF

            @pl.when(nc < NCH)
            def _prefetch():
                issue(nc, k)

        return acc

    acc = lax.fori_loop(
        0, NCH // NBUF, outer, jnp.zeros((CROWS, 1), jnp.float32)
    )
    out_ref[0, 0] = jnp.sum(sc_ref[...]) - EPS * jnp.sum(acc)


def kernel(x, target):
    mask = (target != PAD).astype(jnp.float32).reshape(N_ROWS, 1)
    sc_part = _sc_gather()(x.reshape(-1), target)
    out = pl.pallas_call(
        _tc_body,
        in_specs=[
            pl.BlockSpec(memory_space=pl.ANY),
            pl.BlockSpec(memory_space=pltpu.VMEM),
            pl.BlockSpec(memory_space=pltpu.VMEM),
        ],
        out_specs=pl.BlockSpec(memory_space=pltpu.SMEM),
        out_shape=jax.ShapeDtypeStruct((1, 1), jnp.float32),
        scratch_shapes=[
            pltpu.VMEM((NBUF, CROWS, SIZE), jnp.float32),
            pltpu.SemaphoreType.DMA((NBUF,)),
        ],
    )(x, mask, sc_part)
    return out.reshape(())
